# Initial kernel scaffold; baseline (speedup 1.0000x reference)
import jax
import jax.numpy as jnp
from jax.experimental import pallas as pl


def kernel(lr, start_w, start_b, bottom_w, bottom_b, end_w, end_b, down0_w, down0_b, down1_w, down1_b, up0_w, up0_b, up1_w, up1_b, pool0_w, pool0_b, pool1_w, pool1_b, gsr_weights, res0_w1, res0_w2, res1_w1, res1_w2):
    raise NotImplementedError("write your pallas kernel here")


# speedup vs baseline: 1.0128x; 1.0128x over previous
from functools import partial

import jax
import jax.numpy as jnp
from jax.experimental import pallas as pl
from jax.experimental.pallas import tpu as pltpu

F32 = jnp.float32

_LR = 128
_HR = 256
_D = 256
_K1 = 128
_K2 = 64
_NUM_RES = 2


def _mm(a, b):
    return jnp.dot(a, b, preferred_element_type=F32)


def _mm_t(a, b):
    return jax.lax.dot_general(a, b, (((1,), (1,)), ((), ())),
                               preferred_element_type=F32)


def _diag_mask(n):
    r = jax.lax.broadcasted_iota(jnp.int32, (n, n), 0)
    c = jax.lax.broadcasted_iota(jnp.int32, (n, n), 1)
    return r == c


def _rank_select(scores_col, n, kk):
    scores_row = scores_col.T
    r = jax.lax.broadcasted_iota(jnp.int32, (n, n), 0)
    c = jax.lax.broadcasted_iota(jnp.int32, (n, n), 1)
    ahead = ((scores_row > scores_col)
             | ((scores_row == scores_col) & (c < r))).astype(F32)
    rank_col = jnp.sum(ahead, axis=1, keepdims=True)
    rank_row = rank_col.T
    p = (jax.lax.broadcasted_iota(jnp.int32, (kk, n), 0)
         == rank_row.astype(jnp.int32)).astype(F32)
    pt = (jax.lax.broadcasted_iota(jnp.int32, (n, kk), 1)
          == rank_col.astype(jnp.int32)).astype(F32)
    vals = jnp.sum(p * scores_row, axis=1, keepdims=True)
    return p, pt, vals


def _fused_body(G, adj_ref, ut_ref, sw_ref, gw_ref, gb_ref, pw_ref, pb_ref,
                ew_ref, eb_ref, am_ref, r1_ref, r2_ref,
                z_ref, adj_out_ref, net_ref, sg_ref):
    N, K1, K2, D = _LR, _K1, _K2, _D

    adj = adj_ref[...]
    A0 = [adj[g] for g in range(G)]

    X = _mm(adj.reshape(G * N, N), sw_ref[...]) + gb_ref[0]
    sg_ref[...] = X.reshape(G, N, D)
    org = X

    def gcn(A_list, X_all, n, w, b):
        ax = jnp.concatenate(
            [_mm(A_list[g], X_all[g * n:(g + 1) * n]) for g in range(G)], 0)
        return _mm(ax, w) + b

    def pool(X_all, n, kk, A_list, pw, pb):
        s = jnp.sum(X_all * pw, axis=1, keepdims=True) + pb
        sc = jax.nn.sigmoid(s / 100.0)
        Xp, Ap, PTs = [], [], []
        for g in range(G):
            p, pt, v = _rank_select(sc[g * n:(g + 1) * n], n, kk)
            Xp.append(_mm(p, X_all[g * n:(g + 1) * n]) * v)
            Ap.append(_mm(_mm(p, A_list[g]), pt))
            PTs.append(pt)
        return jnp.concatenate(Xp, 0), Ap, PTs

    X = gcn(A0, X, N, gw_ref[0], gb_ref[1])
    down0 = X
    X, A1, PT0 = pool(X, N, K1, A0, pw_ref[0], pb_ref[0])

    X = gcn(A1, X, K1, gw_ref[1], gb_ref[2])
    down1 = X
    X, A2, PT1 = pool(X, K1, K2, A1, pw_ref[1], pb_ref[1])

    X = gcn(A2, X, K2, gw_ref[2], gb_ref[3])

    X = jnp.concatenate(
        [_mm(PT1[g], X[g * K2:(g + 1) * K2]) for g in range(G)], 0)
    X = gcn(A1, X, K1, gw_ref[3], gb_ref[4]) + down1

    X = jnp.concatenate(
        [_mm(PT0[g], X[g * K1:(g + 1) * K1]) for g in range(G)], 0)
    X = gcn(A0, X, N, gw_ref[4], gb_ref[5]) + down0

    ax = jnp.concatenate(
        [_mm(A0[g], X[g * N:(g + 1) * N]) for g in range(G)], 0)
    aorg = jnp.concatenate(
        [_mm(A0[g], org[g * N:(g + 1) * N]) for g in range(G)], 0)
    net = _mm(ax, ew_ref[0]) + _mm(aorg, ew_ref[1]) + eb_ref[...]
    net_ref[...] = net.reshape(G, N, _HR)

    am = am_ref[...]
    eye = _diag_mask(_HR)
    adj_outs, hs = [], []
    for g in range(G):
        bg = _mm(am, ut_ref[g])
        fd = jnp.abs(_mm(bg, net[g * N:(g + 1) * N]))
        ao = jnp.where(eye, 1.0, fd)
        adj_outs.append(ao)
        xg = _mm_t(ao, ao)
        hs.append(jnp.abs(jnp.where(eye, 1.0, xg)))
    adj_out_ref[...] = jnp.stack(adj_outs)

    H = jnp.concatenate(hs, 0)
    for i in range(_NUM_RES):
        t = _mm(H, r1_ref[i])
        h1 = jnp.concatenate(
            [jnp.maximum(_mm(adj_outs[g], t[g * _HR:(g + 1) * _HR]), 0.0)
             for g in range(G)], 0)
        t = _mm(h1, r2_ref[i])
        h2 = jnp.concatenate(
            [jnp.maximum(_mm(adj_outs[g], t[g * _HR:(g + 1) * _HR]), 0.0)
             for g in range(G)], 0)
        H = h2 + H

    zs = []
    for g in range(G):
        h = H[g * _HR:(g + 1) * _HR]
        z = (h + h.T) * 0.5
        zs.append(jnp.abs(jnp.where(eye, 1.0, z)))
    z_ref[...] = jnp.stack(zs)


def kernel(lr, start_w, start_b, bottom_w, bottom_b, end_w, end_b,
           down0_w, down0_b, down1_w, down1_b,
           up0_w, up0_b, up1_w, up1_b,
           pool0_w, pool0_b, pool1_w, pool1_b,
           gsr_weights, res0_w1, res0_w2, res1_w1, res1_w2):
    lr = jnp.asarray(lr, F32)
    squeeze = lr.ndim == 2
    if squeeze:
        lr = lr[None]
    batch = lr.shape[0]

    rowsum = jnp.sum(lr, axis=-1, keepdims=True)
    r = jnp.where(rowsum == 0.0, 0.0, jax.lax.rsqrt(rowsum))
    adj_mx = jnp.swapaxes(lr, -1, -2) * r * jnp.swapaxes(r, -1, -2)

    _, u = jnp.linalg.eigh(adj_mx)
    ut = jnp.swapaxes(u, -1, -2)

    gcn_w = jnp.stack([down0_w, down1_w, bottom_w, up0_w, up1_w])
    gcn_b = jnp.stack([start_b, down0_b, down1_b, bottom_b, up0_b, up1_b])
    pool_w = jnp.stack([pool0_w.T, pool1_w.T])
    pool_b = jnp.stack([pool0_b, pool1_b])
    end_w2 = jnp.stack([end_w[:_D], end_w[_D:]])
    res_w1 = jnp.stack([res0_w1, res1_w1])
    res_w2 = jnp.stack([res0_w2, res1_w2])
    a_mat = gsr_weights[:, :_LR] + gsr_weights[:, _LR:]

    G = 4 if batch % 4 == 0 else (2 if batch % 2 == 0 else 1)

    def _per_graph(arr):
        tail = arr.shape[1:]
        return pl.BlockSpec((G,) + tail,
                            lambda b, _nd=len(tail): (b,) + (0,) * _nd)

    def _shared(arr):
        return pl.BlockSpec(arr.shape, lambda b, _nd=arr.ndim: (0,) * _nd)

    inputs = (adj_mx, ut, start_w, gcn_w, gcn_b, pool_w, pool_b,
              end_w2, end_b, a_mat, res_w1, res_w2)
    in_specs = [_per_graph(adj_mx), _per_graph(ut)]
    in_specs += [_shared(a) for a in inputs[2:]]

    out_shape = (
        jax.ShapeDtypeStruct((batch, _HR, _HR), F32),
        jax.ShapeDtypeStruct((batch, _HR, _HR), F32),
        jax.ShapeDtypeStruct((batch, _LR, _HR), F32),
        jax.ShapeDtypeStruct((batch, _LR, _D), F32),
    )
    out_specs = tuple(
        pl.BlockSpec((G,) + s.shape[1:], lambda b: (b, 0, 0))
        for s in out_shape)

    abs_z, outputs, net_outs, start_gcn_outs = pl.pallas_call(
        partial(_fused_body, G),
        grid=(batch // G,),
        out_shape=out_shape,
        in_specs=in_specs,
        out_specs=out_specs,
        compiler_params=pltpu.CompilerParams(
            dimension_semantics=("parallel",)),
    )(*inputs)

    if squeeze:
        abs_z, net_outs, start_gcn_outs, outputs = (
            abs_z[0], net_outs[0], start_gcn_outs[0], outputs[0])
    return abs_z, net_outs, start_gcn_outs, outputs


# speedup vs baseline: 2.0239x; 1.9983x over previous
from functools import partial

import numpy as np

import jax
import jax.numpy as jnp
from jax.experimental import pallas as pl
from jax.experimental.pallas import tpu as pltpu
from jax.sharding import Mesh, PartitionSpec

F32 = jnp.float32

_LR = 128
_HR = 256
_D = 256
_K1 = 128
_K2 = 64
_NUM_RES = 2


def _mm(a, b):
    return jnp.dot(a, b, preferred_element_type=F32)


def _mm_t(a, b):
    return jax.lax.dot_general(a, b, (((1,), (1,)), ((), ())),
                               preferred_element_type=F32)


def _diag_mask(n):
    r = jax.lax.broadcasted_iota(jnp.int32, (n, n), 0)
    c = jax.lax.broadcasted_iota(jnp.int32, (n, n), 1)
    return r == c


def _rank_select(scores_col, n, kk):
    scores_row = scores_col.T
    r = jax.lax.broadcasted_iota(jnp.int32, (n, n), 0)
    c = jax.lax.broadcasted_iota(jnp.int32, (n, n), 1)
    ahead = ((scores_row > scores_col)
             | ((scores_row == scores_col) & (c < r))).astype(F32)
    rank_col = jnp.sum(ahead, axis=1, keepdims=True)
    rank_row = rank_col.T
    p = (jax.lax.broadcasted_iota(jnp.int32, (kk, n), 0)
         == rank_row.astype(jnp.int32)).astype(F32)
    pt = (jax.lax.broadcasted_iota(jnp.int32, (n, kk), 1)
          == rank_col.astype(jnp.int32)).astype(F32)
    vals = jnp.sum(p * scores_row, axis=1, keepdims=True)
    return p, pt, vals


def _fused_body(G, adj_ref, ut_ref, sw_ref, gw_ref, gb_ref, pw_ref, pb_ref,
                ew_ref, eb_ref, am_ref, r1_ref, r2_ref,
                z_ref, adj_out_ref, net_ref, sg_ref):
    N, K1, K2, D = _LR, _K1, _K2, _D

    adj = adj_ref[...]
    A0 = [adj[g] for g in range(G)]

    X = _mm(adj.reshape(G * N, N), sw_ref[...]) + gb_ref[0]
    sg_ref[...] = X.reshape(G, N, D)
    org = X

    def gcn(A_list, X_all, n, w, b):
        ax = jnp.concatenate(
            [_mm(A_list[g], X_all[g * n:(g + 1) * n]) for g in range(G)], 0)
        return _mm(ax, w) + b

    def pool(X_all, n, kk, A_list, pw, pb):
        s = jnp.sum(X_all * pw, axis=1, keepdims=True) + pb
        sc = jax.nn.sigmoid(s / 100.0)
        Xp, Ap, PTs = [], [], []
        for g in range(G):
            p, pt, v = _rank_select(sc[g * n:(g + 1) * n], n, kk)
            Xp.append(_mm(p, X_all[g * n:(g + 1) * n]) * v)
            Ap.append(_mm(_mm(p, A_list[g]), pt))
            PTs.append(pt)
        return jnp.concatenate(Xp, 0), Ap, PTs

    X = gcn(A0, X, N, gw_ref[0], gb_ref[1])
    down0 = X
    X, A1, PT0 = pool(X, N, K1, A0, pw_ref[0], pb_ref[0])

    X = gcn(A1, X, K1, gw_ref[1], gb_ref[2])
    down1 = X
    X, A2, PT1 = pool(X, K1, K2, A1, pw_ref[1], pb_ref[1])

    X = gcn(A2, X, K2, gw_ref[2], gb_ref[3])

    X = jnp.concatenate(
        [_mm(PT1[g], X[g * K2:(g + 1) * K2]) for g in range(G)], 0)
    X = gcn(A1, X, K1, gw_ref[3], gb_ref[4]) + down1

    X = jnp.concatenate(
        [_mm(PT0[g], X[g * K1:(g + 1) * K1]) for g in range(G)], 0)
    X = gcn(A0, X, N, gw_ref[4], gb_ref[5]) + down0

    ax = jnp.concatenate(
        [_mm(A0[g], X[g * N:(g + 1) * N]) for g in range(G)], 0)
    aorg = jnp.concatenate(
        [_mm(A0[g], org[g * N:(g + 1) * N]) for g in range(G)], 0)
    net = _mm(ax, ew_ref[0]) + _mm(aorg, ew_ref[1]) + eb_ref[...]
    net_ref[...] = net.reshape(G, N, _HR)

    am = am_ref[...]
    eye = _diag_mask(_HR)
    adj_outs, hs = [], []
    for g in range(G):
        bg = _mm(am, ut_ref[g])
        fd = jnp.abs(_mm(bg, net[g * N:(g + 1) * N]))
        ao = jnp.where(eye, 1.0, fd)
        adj_outs.append(ao)
        xg = _mm_t(ao, ao)
        hs.append(jnp.abs(jnp.where(eye, 1.0, xg)))
    adj_out_ref[...] = jnp.stack(adj_outs)

    H = jnp.concatenate(hs, 0)
    for i in range(_NUM_RES):
        t = _mm(H, r1_ref[i])
        h1 = jnp.concatenate(
            [jnp.maximum(_mm(adj_outs[g], t[g * _HR:(g + 1) * _HR]), 0.0)
             for g in range(G)], 0)
        t = _mm(h1, r2_ref[i])
        h2 = jnp.concatenate(
            [jnp.maximum(_mm(adj_outs[g], t[g * _HR:(g + 1) * _HR]), 0.0)
             for g in range(G)], 0)
        H = h2 + H

    zs = []
    for g in range(G):
        h = H[g * _HR:(g + 1) * _HR]
        z = (h + h.T) * 0.5
        zs.append(jnp.abs(jnp.where(eye, 1.0, z)))
    z_ref[...] = jnp.stack(zs)


def _pipeline(lr, start_w, start_b, bottom_w, bottom_b, end_w, end_b,
              down0_w, down0_b, down1_w, down1_b,
              up0_w, up0_b, up1_w, up1_b,
              pool0_w, pool0_b, pool1_w, pool1_b,
              gsr_weights, res0_w1, res0_w2, res1_w1, res1_w2):
    batch = lr.shape[0]

    rowsum = jnp.sum(lr, axis=-1, keepdims=True)
    r = jnp.where(rowsum == 0.0, 0.0, jax.lax.rsqrt(rowsum))
    adj_mx = jnp.swapaxes(lr, -1, -2) * r * jnp.swapaxes(r, -1, -2)

    _, u = jnp.linalg.eigh(adj_mx)
    ut = jnp.swapaxes(u, -1, -2)

    gcn_w = jnp.stack([down0_w, down1_w, bottom_w, up0_w, up1_w])
    gcn_b = jnp.stack([start_b, down0_b, down1_b, bottom_b, up0_b, up1_b])
    pool_w = jnp.stack([pool0_w.T, pool1_w.T])
    pool_b = jnp.stack([pool0_b, pool1_b])
    end_w2 = jnp.stack([end_w[:_D], end_w[_D:]])
    res_w1 = jnp.stack([res0_w1, res1_w1])
    res_w2 = jnp.stack([res0_w2, res1_w2])
    a_mat = gsr_weights[:, :_LR] + gsr_weights[:, _LR:]

    G = 4 if batch % 4 == 0 else (2 if batch % 2 == 0 else 1)

    def _per_graph(arr):
        tail = arr.shape[1:]
        return pl.BlockSpec((G,) + tail,
                            lambda b, _nd=len(tail): (b,) + (0,) * _nd)

    def _shared(arr):
        return pl.BlockSpec(arr.shape, lambda b, _nd=arr.ndim: (0,) * _nd)

    inputs = (adj_mx, ut, start_w, gcn_w, gcn_b, pool_w, pool_b,
              end_w2, end_b, a_mat, res_w1, res_w2)
    in_specs = [_per_graph(adj_mx), _per_graph(ut)]
    in_specs += [_shared(a) for a in inputs[2:]]

    out_shape = (
        jax.ShapeDtypeStruct((batch, _HR, _HR), F32),
        jax.ShapeDtypeStruct((batch, _HR, _HR), F32),
        jax.ShapeDtypeStruct((batch, _LR, _HR), F32),
        jax.ShapeDtypeStruct((batch, _LR, _D), F32),
    )
    out_specs = tuple(
        pl.BlockSpec((G,) + s.shape[1:], lambda b: (b, 0, 0))
        for s in out_shape)

    abs_z, outputs, net_outs, start_gcn_outs = pl.pallas_call(
        partial(_fused_body, G),
        grid=(batch // G,),
        out_shape=out_shape,
        in_specs=in_specs,
        out_specs=out_specs,
        compiler_params=pltpu.CompilerParams(
            dimension_semantics=("parallel",)),
    )(*inputs)

    return abs_z, net_outs, start_gcn_outs, outputs


def kernel(lr, *params):
    lr = jnp.asarray(lr, F32)
    squeeze = lr.ndim == 2
    if squeeze:
        lr = lr[None]
    batch = lr.shape[0]

    devs = jax.devices()
    n_dev = len(devs)
    while n_dev > 1 and batch % n_dev != 0:
        n_dev -= 1
    if n_dev > 1:
        mesh = Mesh(np.array(devs[:n_dev]), ("b",))
        spec_in = (PartitionSpec("b"),) + (PartitionSpec(),) * len(params)
        fn = jax.shard_map(_pipeline, mesh=mesh, in_specs=spec_in,
                           out_specs=PartitionSpec("b"), check_vma=False)
        abs_z, net_outs, start_gcn_outs, outputs = fn(lr, *params)
    else:
        abs_z, net_outs, start_gcn_outs, outputs = _pipeline(lr, *params)

    if squeeze:
        abs_z, net_outs, start_gcn_outs, outputs = (
            abs_z[0], net_outs[0], start_gcn_outs[0], outputs[0])
    return abs_z, net_outs, start_gcn_outs, outputs


# speedup vs baseline: 2.0254x; 1.0008x over previous
from functools import partial

import numpy as np

import jax
import jax.numpy as jnp
from jax.experimental import pallas as pl
from jax.experimental.pallas import tpu as pltpu
from jax.sharding import Mesh, PartitionSpec

F32 = jnp.float32

_LR = 128
_HR = 256
_D = 256
_K1 = 128
_K2 = 64
_NUM_RES = 2


def _mm(a, b):
    return jnp.dot(a, b, preferred_element_type=F32)


def _mm_t(a, b):
    return jax.lax.dot_general(a, b, (((1,), (1,)), ((), ())),
                               preferred_element_type=F32)


def _diag_mask(n):
    r = jax.lax.broadcasted_iota(jnp.int32, (n, n), 0)
    c = jax.lax.broadcasted_iota(jnp.int32, (n, n), 1)
    return r == c


def _rank_select(scores_col, n, kk):
    scores_row = scores_col.T
    r = jax.lax.broadcasted_iota(jnp.int32, (n, n), 0)
    c = jax.lax.broadcasted_iota(jnp.int32, (n, n), 1)
    ahead = ((scores_row > scores_col)
             | ((scores_row == scores_col) & (c < r))).astype(F32)
    rank_col = jnp.sum(ahead, axis=1, keepdims=True)
    rank_row = rank_col.T
    p = (jax.lax.broadcasted_iota(jnp.int32, (kk, n), 0)
         == rank_row.astype(jnp.int32)).astype(F32)
    pt = (jax.lax.broadcasted_iota(jnp.int32, (n, kk), 1)
          == rank_col.astype(jnp.int32)).astype(F32)
    vals = jnp.sum(p * scores_row, axis=1, keepdims=True)
    return p, pt, vals


def _fused_body(G, adj_ref, ut_ref, sw_ref, gw_ref, gb_ref, pw_ref, pb_ref,
                ew_ref, eb_ref, am_ref, r1_ref, r2_ref,
                z_ref, adj_out_ref, net_ref, sg_ref):
    N, K1, K2, D = _LR, _K1, _K2, _D

    adj = adj_ref[...]
    A0 = [adj[g] for g in range(G)]

    X = _mm(adj.reshape(G * N, N), sw_ref[...]) + gb_ref[0]
    sg_ref[...] = X.reshape(G, N, D)
    org = X

    def gcn(A_list, X_all, n, w, b):
        ax = jnp.concatenate(
            [_mm(A_list[g], X_all[g * n:(g + 1) * n]) for g in range(G)], 0)
        return _mm(ax, w) + b

    def pool(X_all, n, kk, A_list, pw, pb):
        s = jnp.sum(X_all * pw, axis=1, keepdims=True) + pb
        sc = jax.nn.sigmoid(s / 100.0)
        Xp, Ap, PTs = [], [], []
        for g in range(G):
            p, pt, v = _rank_select(sc[g * n:(g + 1) * n], n, kk)
            Xp.append(_mm(p, X_all[g * n:(g + 1) * n]) * v)
            Ap.append(_mm(_mm(p, A_list[g]), pt))
            PTs.append(pt)
        return jnp.concatenate(Xp, 0), Ap, PTs

    X = gcn(A0, X, N, gw_ref[0], gb_ref[1])
    down0 = X
    X, A1, PT0 = pool(X, N, K1, A0, pw_ref[0], pb_ref[0])

    X = gcn(A1, X, K1, gw_ref[1], gb_ref[2])
    down1 = X
    X, A2, PT1 = pool(X, K1, K2, A1, pw_ref[1], pb_ref[1])

    X = gcn(A2, X, K2, gw_ref[2], gb_ref[3])

    X = jnp.concatenate(
        [_mm(PT1[g], X[g * K2:(g + 1) * K2]) for g in range(G)], 0)
    X = gcn(A1, X, K1, gw_ref[3], gb_ref[4]) + down1

    X = jnp.concatenate(
        [_mm(PT0[g], X[g * K1:(g + 1) * K1]) for g in range(G)], 0)
    X = gcn(A0, X, N, gw_ref[4], gb_ref[5]) + down0

    ax = jnp.concatenate(
        [_mm(A0[g], X[g * N:(g + 1) * N]) for g in range(G)], 0)
    aorg = jnp.concatenate(
        [_mm(A0[g], org[g * N:(g + 1) * N]) for g in range(G)], 0)
    net = _mm(ax, ew_ref[0]) + _mm(aorg, ew_ref[1]) + eb_ref[...]
    net_ref[...] = net.reshape(G, N, _HR)

    am = am_ref[...]
    eye = _diag_mask(_HR)
    adj_outs, hs = [], []
    for g in range(G):
        bg = _mm(am, ut_ref[g])
        fd = jnp.abs(_mm(bg, net[g * N:(g + 1) * N]))
        ao = jnp.where(eye, 1.0, fd)
        adj_outs.append(ao)
        xg = _mm_t(ao, ao)
        hs.append(jnp.abs(jnp.where(eye, 1.0, xg)))
    adj_out_ref[...] = jnp.stack(adj_outs)

    H = jnp.concatenate(hs, 0)
    for i in range(_NUM_RES):
        t = _mm(H, r1_ref[i])
        h1 = jnp.concatenate(
            [jnp.maximum(_mm(adj_outs[g], t[g * _HR:(g + 1) * _HR]), 0.0)
             for g in range(G)], 0)
        t = _mm(h1, r2_ref[i])
        h2 = jnp.concatenate(
            [jnp.maximum(_mm(adj_outs[g], t[g * _HR:(g + 1) * _HR]), 0.0)
             for g in range(G)], 0)
        H = h2 + H

    zs = []
    for g in range(G):
        h = H[g * _HR:(g + 1) * _HR]
        z = (h + h.T) * 0.5
        zs.append(jnp.abs(jnp.where(eye, 1.0, z)))
    z_ref[...] = jnp.stack(zs)


def _pipeline(lr, start_w, start_b, bottom_w, bottom_b, end_w, end_b,
              down0_w, down0_b, down1_w, down1_b,
              up0_w, up0_b, up1_w, up1_b,
              pool0_w, pool0_b, pool1_w, pool1_b,
              gsr_weights, res0_w1, res0_w2, res1_w1, res1_w2):
    batch = lr.shape[0]

    rowsum = jnp.sum(lr, axis=-1, keepdims=True)
    r = jnp.where(rowsum == 0.0, 0.0, jax.lax.rsqrt(rowsum))
    adj_mx = jnp.swapaxes(lr, -1, -2) * r * jnp.swapaxes(r, -1, -2)

    _, u = jnp.linalg.eigh(adj_mx)
    ut = jnp.swapaxes(u, -1, -2)

    gcn_w = jnp.stack([down0_w, down1_w, bottom_w, up0_w, up1_w])
    gcn_b = jnp.stack([start_b, down0_b, down1_b, bottom_b, up0_b, up1_b])
    pool_w = jnp.stack([pool0_w.T, pool1_w.T])
    pool_b = jnp.stack([pool0_b, pool1_b])
    end_w2 = jnp.stack([end_w[:_D], end_w[_D:]])
    res_w1 = jnp.stack([res0_w1, res1_w1])
    res_w2 = jnp.stack([res0_w2, res1_w2])
    a_mat = gsr_weights[:, :_LR] + gsr_weights[:, _LR:]

    G = 8 if batch % 8 == 0 else (4 if batch % 4 == 0 else
                                  (2 if batch % 2 == 0 else 1))

    def _per_graph(arr):
        tail = arr.shape[1:]
        return pl.BlockSpec((G,) + tail,
                            lambda b, _nd=len(tail): (b,) + (0,) * _nd)

    def _shared(arr):
        return pl.BlockSpec(arr.shape, lambda b, _nd=arr.ndim: (0,) * _nd)

    inputs = (adj_mx, ut, start_w, gcn_w, gcn_b, pool_w, pool_b,
              end_w2, end_b, a_mat, res_w1, res_w2)
    in_specs = [_per_graph(adj_mx), _per_graph(ut)]
    in_specs += [_shared(a) for a in inputs[2:]]

    out_shape = (
        jax.ShapeDtypeStruct((batch, _HR, _HR), F32),
        jax.ShapeDtypeStruct((batch, _HR, _HR), F32),
        jax.ShapeDtypeStruct((batch, _LR, _HR), F32),
        jax.ShapeDtypeStruct((batch, _LR, _D), F32),
    )
    out_specs = tuple(
        pl.BlockSpec((G,) + s.shape[1:], lambda b: (b, 0, 0))
        for s in out_shape)

    abs_z, outputs, net_outs, start_gcn_outs = pl.pallas_call(
        partial(_fused_body, G),
        grid=(batch // G,),
        out_shape=out_shape,
        in_specs=in_specs,
        out_specs=out_specs,
        compiler_params=pltpu.CompilerParams(
            dimension_semantics=("parallel",)),
    )(*inputs)

    return abs_z, net_outs, start_gcn_outs, outputs


def kernel(lr, *params):
    lr = jnp.asarray(lr, F32)
    squeeze = lr.ndim == 2
    if squeeze:
        lr = lr[None]
    batch = lr.shape[0]

    devs = jax.devices()
    n_dev = len(devs)
    while n_dev > 1 and batch % n_dev != 0:
        n_dev -= 1
    if n_dev > 1:
        mesh = Mesh(np.array(devs[:n_dev]), ("b",))
        spec_in = (PartitionSpec("b"),) + (PartitionSpec(),) * len(params)
        fn = jax.shard_map(_pipeline, mesh=mesh, in_specs=spec_in,
                           out_specs=PartitionSpec("b"), check_vma=False)
        abs_z, net_outs, start_gcn_outs, outputs = fn(lr, *params)
    else:
        abs_z, net_outs, start_gcn_outs, outputs = _pipeline(lr, *params)

    if squeeze:
        abs_z, net_outs, start_gcn_outs, outputs = (
            abs_z[0], net_outs[0], start_gcn_outs[0], outputs[0])
    return abs_z, net_outs, start_gcn_outs, outputs
